# baseline (speedup 1.0000x reference)
import functools

import jax
import jax.numpy as jnp
from jax.experimental import pallas as pl
from jax.experimental.pallas import tpu as pltpu

NEG = -1e30


def _lstm_decode_body(xp_ref, start_ref, ct_ref, h0_ref, wih_ref, whh_ref,
                      se_ref, bih_ref, bhh_ref, gate_ref, bias_ref,
                      wdec_ref, bdec_ref,
                      lls_ref, eop_ref, h_scr, c_scr, bk_scr,
                      wihx_scr, whht_scr, wdec_scr,
                      *, eop_idx, bsz):
    t = pl.program_id(1)
    KB = gate_ref.shape[0]
    E = start_ref.shape[1]
    BB = KB * ct_ref.shape[2]
    H = whh_ref.shape[1]
    R = BB // KB
    V = wdec_ref.shape[1]

    @pl.when(t == 0)
    def _():
        h_scr[...] = jnp.broadcast_to(
            jnp.tanh(h0_ref[:, 0:H]), (BB, H)).astype(jnp.bfloat16)
        c_scr[...] = jnp.broadcast_to(h0_ref[:, H:2 * H], (BB, H))
        bk_scr[...] = (bih_ref[...] + bhh_ref[...] +
                       jax.lax.dot_general(
                           se_ref[...], wih_ref[:, E:],
                           (((1,), (1,)), ((), ())),
                           preferred_element_type=jnp.float32))
        wihx_scr[...] = jnp.transpose(wih_ref[:, 0:E]).astype(jnp.bfloat16)
        whht_scr[...] = jnp.transpose(whh_ref[...]).astype(jnp.bfloat16)
        wdec_scr[...] = wdec_ref[...].astype(jnp.bfloat16)

    off = pl.multiple_of(jnp.maximum(t - 1, 0) * bsz, bsz)
    xw = xp_ref[pl.ds(off, R), :]
    x = jnp.where(t == 0, jnp.broadcast_to(start_ref[...], (R, E)), xw)

    xg = jnp.dot(x.astype(jnp.bfloat16), wihx_scr[...],
                 preferred_element_type=jnp.float32)
    h = h_scr[...]
    hg = jnp.dot(h, whht_scr[...],
                 preferred_element_type=jnp.float32)

    xgb = jnp.broadcast_to(xg[None], (KB, R, 4 * H)).reshape(BB, 4 * H)
    bkb = jnp.broadcast_to(bk_scr[...][:, None, :], (KB, R, 4 * H)).reshape(BB, 4 * H)
    gates = hg + xgb + bkb

    i = jax.nn.sigmoid(gates[:, 0:H])
    f = jax.nn.sigmoid(gates[:, H:2 * H])
    g = jnp.tanh(gates[:, 2 * H:3 * H])
    o = jax.nn.sigmoid(gates[:, 3 * H:4 * H])
    c_new = f * c_scr[...] + i * g
    h_new = o * jnp.tanh(c_new)
    h_scr[...] = h_new.astype(jnp.bfloat16)
    c_scr[...] = c_new

    gmul = jnp.broadcast_to(gate_ref[...][:, None, :], (KB, R, H)).reshape(BB, H)
    badd = jnp.broadcast_to(bias_ref[...][:, None, :], (KB, R, H)).reshape(BB, H)
    s = (gmul * h_new + badd).astype(jnp.bfloat16)
    logits = jnp.dot(s, wdec_scr[...], preferred_element_type=jnp.float32) + bdec_ref[...]

    ctrow = ct_ref[0]
    mask = (jax.lax.broadcasted_iota(jnp.int32, (V, R), 0)
            == jnp.broadcast_to(ctrow, (V, R)))

    lls_rows, eop_rows = [], []
    for k in range(KB):
        tk = jnp.transpose(logits[k * R:(k + 1) * R, :])
        mx = jnp.max(tk, axis=0, keepdims=True)
        lse = jnp.log(jnp.sum(jnp.exp(tk - mx), axis=0, keepdims=True)) + mx
        lls_rows.append(jnp.sum(jnp.where(mask, tk, 0.0), axis=0, keepdims=True) - lse)
        eop_rows.append(tk[eop_idx:eop_idx + 1, :] - lse)
    lls_ref[0] = jnp.concatenate(lls_rows, axis=0)
    eop_ref[0] = jnp.concatenate(eop_rows, axis=0)


def _lstm_decode(xp, start_row, ct3, h0_row, wih, whh, se2d, bih, bhh,
                 gates_k, biases_k, wdec_pad, bdec_pad, nb, eop_idx, Lp1, bsz):
    K, H = gates_k.shape
    E = start_row.shape[1]
    V = wdec_pad.shape[1]
    L, _, R = ct3.shape
    KB = K // nb
    body = functools.partial(_lstm_decode_body, eop_idx=eop_idx, bsz=bsz)
    return pl.pallas_call(
        body,
        out_shape=(jax.ShapeDtypeStruct((Lp1, K, R), jnp.float32),
                   jax.ShapeDtypeStruct((Lp1, K, R), jnp.float32)),
        grid_spec=pltpu.PrefetchScalarGridSpec(
            num_scalar_prefetch=0,
            grid=(nb, Lp1),
            in_specs=[
                pl.BlockSpec(xp.shape, lambda n, t: (0, 0)),
                pl.BlockSpec((1, E), lambda n, t: (0, 0)),
                pl.BlockSpec((1, 1, R), lambda n, t: (jnp.minimum(t, L - 1), 0, 0)),
                pl.BlockSpec((1, 2 * H), lambda n, t: (0, 0)),
                pl.BlockSpec(wih.shape, lambda n, t: (0, 0)),
                pl.BlockSpec(whh.shape, lambda n, t: (0, 0)),
                pl.BlockSpec((KB, se2d.shape[1]), lambda n, t: (n, 0)),
                pl.BlockSpec((1, 4 * H), lambda n, t: (0, 0)),
                pl.BlockSpec((1, 4 * H), lambda n, t: (0, 0)),
                pl.BlockSpec((KB, H), lambda n, t: (n, 0)),
                pl.BlockSpec((KB, H), lambda n, t: (n, 0)),
                pl.BlockSpec((H, V), lambda n, t: (0, 0)),
                pl.BlockSpec((1, V), lambda n, t: (0, 0)),
            ],
            out_specs=(pl.BlockSpec((1, KB, R), lambda n, t: (t, n, 0)),
                       pl.BlockSpec((1, KB, R), lambda n, t: (t, n, 0))),
            scratch_shapes=[pltpu.VMEM((KB * R, H), jnp.bfloat16),
                            pltpu.VMEM((KB * R, H), jnp.float32),
                            pltpu.VMEM((KB, 4 * H), jnp.float32),
                            pltpu.VMEM((E, 4 * H), jnp.bfloat16),
                            pltpu.VMEM((H, 4 * H), jnp.bfloat16),
                            pltpu.VMEM((H, V), jnp.bfloat16)],
        ),
        compiler_params=pltpu.CompilerParams(
            dimension_semantics=("parallel", "arbitrary")),
    )(xp, start_row, ct3, h0_row, wih, whh, se2d, bih, bhh,
      gates_k, biases_k, wdec_pad, bdec_pad)


def _dp_body(lls_ref, eop_ref, se_ref, tw_ref, tb_ref, lsc_ref, init_ref,
             out_ref, *, L, bsz, seqlen):
    K = se_ref.shape[0]
    T = seqlen

    a = jnp.dot(se_ref[...], tw_ref[...], preferred_element_type=jnp.float32)
    sc = jax.lax.dot_general(a, se_ref[...], (((1,), (1,)), ((), ())),
                             preferred_element_type=jnp.float32)
    ii = jax.lax.broadcasted_iota(jnp.int32, (K, K), 0)
    jj = jax.lax.broadcasted_iota(jnp.int32, (K, K), 1)
    sc = sc + tb_ref[...] + jnp.where(ii == jj, NEG, 0.0)
    mx = jnp.max(sc, axis=1, keepdims=True)
    tsc = sc - mx - jnp.log(jnp.sum(jnp.exp(sc - mx), axis=1, keepdims=True))
    expT = jnp.exp(tsc)

    lsc = lsc_ref[...]
    len_scal = {}
    for s in range(L):
        v = lsc[:, :s + 1]
        m = jnp.max(v, axis=1, keepdims=True)
        ls = v - m - jnp.log(jnp.sum(jnp.exp(v - m), axis=1, keepdims=True))
        for l in range(s + 1):
            len_scal[(s, l)] = ls[0, l]

    vi = init_ref[...]
    mi = jnp.max(vi, axis=1, keepdims=True)
    ils = vi - mi - jnp.log(jnp.sum(jnp.exp(vi - mi), axis=1, keepdims=True))
    pinit = jnp.exp(ils)

    cum = lls_ref[0]
    obs = []
    for l in range(L):
        if l > 0:
            cum = cum + lls_ref[l]
        obs.append(cum + eop_ref[l + 1])

    zeros = jnp.zeros((K, bsz), jnp.float32)
    beta = {T: zeros}
    bs0 = None
    for t in range(T - 1, -1, -1):
        steps = min(L, T - t)
        terms = []
        for l in range(steps):
            b_next = beta.get(t + l + 1, zeros)
            o = obs[l][:, t * bsz:(t + 1) * bsz]
            terms.append(b_next + o + len_scal[(steps - 1, l)])
        if steps == 1:
            bs = terms[0]
        else:
            m = terms[0]
            for tm in terms[1:]:
                m = jnp.maximum(m, tm)
            acc = jnp.exp(terms[0] - m)
            for tm in terms[1:]:
                acc = acc + jnp.exp(tm - m)
            bs = jnp.log(acc) + m
        bs0 = bs
        if t > 0:
            m2 = jnp.max(bs, axis=0, keepdims=True)
            p = jnp.exp(bs - m2)
            beta[t] = jnp.log(
                jnp.dot(expT, p, preferred_element_type=jnp.float32)) + m2

    mf = jnp.max(bs0, axis=0, keepdims=True)
    fin = jnp.log(jnp.dot(pinit, jnp.exp(bs0 - mf),
                          preferred_element_type=jnp.float32)) + mf
    out_ref[...] = jnp.sum(fin, axis=1, keepdims=True)


def _dp_call(lls, eop, se2d, trans_weights, trans_bias, len_scores, init_trans,
             L, bsz, seqlen):
    body = functools.partial(_dp_body, L=L, bsz=bsz, seqlen=seqlen)
    return pl.pallas_call(
        body,
        out_shape=jax.ShapeDtypeStruct((1, 1), jnp.float32),
        in_specs=[pl.BlockSpec(lls.shape, lambda i: (0, 0, 0)),
                  pl.BlockSpec(eop.shape, lambda i: (0, 0, 0)),
                  pl.BlockSpec(se2d.shape, lambda i: (0, 0)),
                  pl.BlockSpec(trans_weights.shape, lambda i: (0, 0)),
                  pl.BlockSpec(trans_bias.shape, lambda i: (0, 0)),
                  pl.BlockSpec(len_scores.shape, lambda i: (0, 0)),
                  pl.BlockSpec(init_trans.shape, lambda i: (0, 0))],
        out_specs=pl.BlockSpec((1, 1), lambda i: (0, 0)),
        compiler_params=pltpu.CompilerParams(
            dimension_semantics=("arbitrary",)),
        grid=(1,),
    )(lls, eop, se2d, trans_weights, trans_bias, len_scores, init_trans)


def kernel(lut, start_emb, pad_emb, state_embs, state_out_gates, state_out_biases,
           h0_lin, wih, whh, b_ih, b_hh, dec_w, dec_b, trans_weights, trans_bias,
           init_trans, len_scores, inps, combotargs):
    K = state_embs.shape[0]
    L = len_scores.shape[1]
    E = start_emb.shape[-1]
    H = whh.shape[1]
    hsmm_emb = state_embs.shape[-1]
    gentypes = dec_w.shape[0] - 1
    VPAD = 128
    bsz, seqlen = inps.shape
    NB = 2

    embs = jnp.take(lut, inps.T, axis=0)
    xp = jnp.concatenate(
        [embs, jnp.broadcast_to(pad_emb.reshape(1, 1, E), (L - 1, bsz, E))],
        axis=0).reshape((seqlen + L - 1) * bsz, E)

    ct3 = jnp.transpose(combotargs, (1, 2, 0)).reshape(L, 1, seqlen * bsz)

    wdec_pad = jnp.zeros((H, VPAD), jnp.float32).at[:, :gentypes + 1].set(dec_w.T)
    bdec_pad = jnp.full((1, VPAD), NEG, jnp.float32).at[:, :gentypes + 1].set(
        dec_b.reshape(1, gentypes + 1))

    se2d = state_embs.reshape(K, hsmm_emb)
    lls, eop = _lstm_decode(
        xp, start_emb.reshape(1, E), ct3, h0_lin.reshape(1, 2 * H),
        wih, whh, se2d, b_ih.reshape(1, 4 * H), b_hh.reshape(1, 4 * H),
        state_out_gates.reshape(K, H), state_out_biases.reshape(K, H),
        wdec_pad, bdec_pad, NB, gentypes, L + 1, bsz)

    out = _dp_call(lls, eop, se2d, trans_weights, trans_bias,
                   len_scores, init_trans, L, bsz, seqlen)
    return out.reshape(())
